# Initial kernel scaffold; baseline (speedup 1.0000x reference)
import functools

import jax
import jax.numpy as jnp
from jax import lax
from jax.experimental import pallas as pl
from jax.experimental.pallas import tpu as pltpu
from jax.experimental.pallas import tpu_sc as plsc

_N = 10000
_E = 160000
_T = 3
_D = 128
_TAU = 3.0
_BETA = 0.5

_L = 16
_NSUB = 16
_NSL = 640
_NP = _NSUB * _NSL
_EC = _E // _NSUB
_ECP = 10240
_ER = _ECP // 128


def _prep_body(x_ref, w1_ref, b1_ref, w2_ref, wg1_ref, wg2_ref, bg1_ref,
               st_ref, sg_ref, c1_ref):
    w12 = jnp.dot(w1_ref[...], w2_ref[...], preferred_element_type=jnp.float32)
    wg12 = jnp.dot(wg1_ref[...], wg2_ref[...],
                   preferred_element_type=jnp.float32)
    rhs = jnp.concatenate([w12, wg12], axis=1)
    out = jnp.dot(x_ref[...], rhs, preferred_element_type=jnp.float32)
    b1w2 = jnp.dot(b1_ref[...], w2_ref[...], preferred_element_type=jnp.float32)
    st_ref[...] = out[:, 0:1] + b1w2[0, 0]
    sg_ref[...] = out[:, 1:2]
    c1_ref[...] = jnp.dot(bg1_ref[...], wg2_ref[...],
                          preferred_element_type=jnp.float32)


def _prep(x2d, W1, b1, W2, Wg1, Wg2, bg1):
    return pl.pallas_call(
        _prep_body,
        out_shape=(
            jax.ShapeDtypeStruct((_T * _N, 1), jnp.float32),
            jax.ShapeDtypeStruct((_T * _N, 1), jnp.float32),
            jax.ShapeDtypeStruct((1, 1), jnp.float32),
        ),
    )(x2d, W1, b1.reshape(1, -1), W2, Wg1, Wg2, bg1.reshape(1, -1))


_ROWS = 400


def _mv_body(ta_ref, v_ref, add_ref, o_ref, *, add_scale):
    acc = jnp.dot(ta_ref[...], v_ref[...], preferred_element_type=jnp.float32)
    o_ref[...] = acc / _TAU + add_ref[...] * add_scale


def _mv_pass(ta, v, add, add_scale, add_blocked):
    nblk = _N // _ROWS
    add_spec = (pl.BlockSpec((_ROWS, _T), lambda i: (i, 0)) if add_blocked
                else pl.BlockSpec(add.shape, lambda i: (0, 0)))
    return pl.pallas_call(
        functools.partial(_mv_body, add_scale=add_scale),
        grid=(nblk,),
        in_specs=[
            pl.BlockSpec((_ROWS, _N), lambda i: (i, 0)),
            pl.BlockSpec((_N, _T), lambda i: (0, 0)),
            add_spec,
        ],
        out_specs=pl.BlockSpec((_ROWS, _T), lambda i: (i, 0)),
        out_shape=jax.ShapeDtypeStruct((_N, _T), jnp.float32),
    )(ta, v, add)


def _rsqrt16(d):
    i = plsc.bitcast(d, jnp.int32)
    i = 0x5F3759DF - lax.shift_right_arithmetic(i, 1)
    y = plsc.bitcast(i, jnp.float32)
    for _ in range(4):
        y = y * (1.5 - 0.5 * d * y * y)
    return jnp.where(d > 0, y, 0.0)


def _gcn_body(r3_hbm, c3_hbm, w4_hbm, sg_hbm, cst_hbm, out_hbm,
              r2d, c2d, w2d, norm2d, vals2d,
              sg_full, dinv_full, xw2_full,
              sl_a, sl_b, sl_c, zer_sl, cst_loc,
              acc_sh, bcast_sh):
    core = lax.axis_index("c")
    sub = lax.axis_index("s")
    nbase = sub * _NSL
    own = pl.ds(nbase, _NSL)
    f32 = jnp.float32

    pltpu.sync_copy(r3_hbm.at[sub], r2d)
    pltpu.sync_copy(c3_hbm.at[sub], c2d)
    pltpu.sync_copy(cst_hbm, cst_loc)
    zeros16 = jnp.zeros((_L,), f32)

    def zer_body(i, cc):
        zer_sl[pl.ds(i * _L, _L)] = zeros16
        return cc
    lax.fori_loop(0, _NSL // _L, zer_body, 0)
    c1v = cst_loc[pl.ds(0, _L)]
    bg2v = cst_loc[pl.ds(_L, _L)]

    def do_t(t):
        pltpu.sync_copy(w4_hbm.at[t, sub], w2d)
        pltpu.sync_copy(sg_hbm.at[t], sg_full)
        pltpu.sync_copy(zer_sl, acc_sh.at[own])
        plsc.subcore_barrier()
        pltpu.sync_copy(w2d, acc_sh.at[c2d], add=True)
        plsc.subcore_barrier()
        pltpu.sync_copy(acc_sh.at[own], sl_a)

        def dinv_body(i, cc):
            d = sl_a[pl.ds(i * _L, _L)] + 1.0
            sl_b[pl.ds(i * _L, _L)] = _rsqrt16(d)
            return cc
        lax.fori_loop(0, _NSL // _L, dinv_body, 0)
        pltpu.sync_copy(zer_sl, acc_sh.at[own])
        pltpu.sync_copy(sl_b, bcast_sh.at[own])
        plsc.subcore_barrier()
        pltpu.sync_copy(bcast_sh, dinv_full)

        def l1_body(j2, cc):
            for k8 in range(8):
                off = pl.ds(k8 * _L, _L)
                r16 = r2d[j2, off]
                c16 = c2d[j2, off]
                w16 = w2d[j2, off]
                dr = plsc.load_gather(dinv_full, [r16])
                dc = plsc.load_gather(dinv_full, [c16])
                nrm = dr * w16 * dc
                norm2d[j2, off] = nrm
                sv = plsc.load_gather(sg_full, [r16])
                vals2d[j2, off] = nrm * sv
            return cc
        lax.fori_loop(0, _ER, l1_body, 0)
        plsc.subcore_barrier()
        pltpu.sync_copy(vals2d, acc_sh.at[c2d], add=True)
        plsc.subcore_barrier()
        pltpu.sync_copy(acc_sh.at[own], sl_a)

        def xw2_body(i, cc):
            sl16 = pl.ds(i * _L, _L)
            dv = sl_b[sl16]
            sg16 = sg_full[pl.ds(nbase + i * _L, _L)]
            sl_c[sl16] = sl_a[sl16] + dv * dv * sg16 + c1v
            return cc
        lax.fori_loop(0, _NSL // _L, xw2_body, 0)
        pltpu.sync_copy(zer_sl, acc_sh.at[own])
        pltpu.sync_copy(sl_c, bcast_sh.at[own])
        plsc.subcore_barrier()
        pltpu.sync_copy(bcast_sh, xw2_full)

        def l2_body(j2, cc):
            for k8 in range(8):
                off = pl.ds(k8 * _L, _L)
                r16 = r2d[j2, off]
                nrm = norm2d[j2, off]
                xv = plsc.load_gather(xw2_full, [r16])
                vals2d[j2, off] = nrm * xv
            return cc
        lax.fori_loop(0, _ER, l2_body, 0)
        plsc.subcore_barrier()
        pltpu.sync_copy(vals2d, acc_sh.at[c2d], add=True)
        plsc.subcore_barrier()
        pltpu.sync_copy(acc_sh.at[own], sl_a)

        def g2_body(i, cc):
            sl16 = pl.ds(i * _L, _L)
            dv = sl_b[sl16]
            x16 = xw2_full[pl.ds(nbase + i * _L, _L)]
            sl_c[sl16] = sl_a[sl16] + dv * dv * x16 + bg2v
            return cc
        lax.fori_loop(0, _NSL // _L, g2_body, 0)
        pltpu.sync_copy(sl_c, out_hbm.at[t, own])
        plsc.subcore_barrier()

    do_t(core)

    @pl.when(core == 0)
    def _():
        do_t(2)


def _gcn(r3, c3, w4, sgp, cst):
    mesh = plsc.VectorSubcoreMesh(core_axis_name="c", subcore_axis_name="s")
    return pl.kernel(
        _gcn_body,
        out_type=jax.ShapeDtypeStruct((_T, _NP), jnp.float32),
        mesh=mesh,
        scratch_types=[
            pltpu.VMEM((_ER, 128), jnp.int32),
            pltpu.VMEM((_ER, 128), jnp.int32),
            pltpu.VMEM((_ER, 128), jnp.float32),
            pltpu.VMEM((_ER, 128), jnp.float32),
            pltpu.VMEM((_ER, 128), jnp.float32),
            pltpu.VMEM((_NP,), jnp.float32),
            pltpu.VMEM((_NP,), jnp.float32),
            pltpu.VMEM((_NP,), jnp.float32),
            pltpu.VMEM((_NSL,), jnp.float32),
            pltpu.VMEM((_NSL,), jnp.float32),
            pltpu.VMEM((_NSL,), jnp.float32),
            pltpu.VMEM((_NSL,), jnp.float32),
            pltpu.VMEM((2 * _L,), jnp.float32),
            pltpu.VMEM_SHARED((_NP,), jnp.float32),
            pltpu.VMEM_SHARED((_NP,), jnp.float32),
        ],
    )(r3, c3, w4, sgp, cst)


def kernel(TSdata, time_adj, edge, edge_attr, W1, b1, W2, b2,
           Wg1, bg1, Wg2, bg2):
    x2d = TSdata.reshape(_T * _N, _D)
    st, sg, c1 = _prep(x2d, W1, b1, W2, Wg1, Wg2, bg1)

    S = st.reshape(_T, _N).T
    sgp = jnp.pad(sg.reshape(_T, _N), ((0, 0), (0, _NP - _N)))

    edge = edge.astype(jnp.int32)
    rp = jnp.pad(edge[0].reshape(_NSUB, _EC), ((0, 0), (0, _ECP - _EC)),
                 constant_values=_NP - 1).reshape(_NSUB, _ER, 128)
    cp = jnp.pad(edge[1].reshape(_NSUB, _EC), ((0, 0), (0, _ECP - _EC)),
                 constant_values=_NP - 1).reshape(_NSUB, _ER, 128)
    wp = jnp.pad(edge_attr.reshape(_T, _NSUB, _EC),
                 ((0, 0), (0, 0), (0, _ECP - _EC))
                 ).reshape(_T, _NSUB, _ER, 128)

    cst = jnp.concatenate([jnp.full((_L,), c1[0, 0], jnp.float32),
                           jnp.full((_L,), bg2[0], jnp.float32)])

    g2 = _gcn(rp, cp, wp, sgp, cst)

    b2row = jnp.broadcast_to(b2.reshape(1, 1), (1, _T)).astype(jnp.float32)
    inner = _mv_pass(time_adj, S, b2row, 1.0, add_blocked=False)
    g2t = g2[:, :_N].T
    dense = _mv_pass(time_adj, inner, g2t, _BETA, add_blocked=True)

    return dense.T


# speedup vs baseline: 36.6477x; 36.6477x over previous
import functools

import jax
import jax.numpy as jnp
from jax import lax
from jax.experimental import pallas as pl
from jax.experimental.pallas import tpu as pltpu
from jax.experimental.pallas import tpu_sc as plsc

_N = 10000
_E = 160000
_T = 3
_D = 128
_TAU = 3.0
_BETA = 0.5

_L = 16
_NSUB = 16
_NSL = 640
_NP = _NSUB * _NSL
_EC = _E // _NSUB
_ECP = 10240
_ER = _ECP // 128


def _prep_body(x_ref, w1_ref, b1_ref, w2_ref, wg1_ref, wg2_ref, bg1_ref,
               st_ref, sg_ref, c1_ref):
    w12 = jnp.dot(w1_ref[...], w2_ref[...], preferred_element_type=jnp.float32)
    wg12 = jnp.dot(wg1_ref[...], wg2_ref[...],
                   preferred_element_type=jnp.float32)
    rhs = jnp.concatenate([w12, wg12], axis=1)
    out = jnp.dot(x_ref[...], rhs, preferred_element_type=jnp.float32)
    b1w2 = jnp.dot(b1_ref[...], w2_ref[...], preferred_element_type=jnp.float32)
    st_ref[...] = out[:, 0:1] + b1w2[0, 0]
    sg_ref[...] = out[:, 1:2]
    c1_ref[...] = jnp.dot(bg1_ref[...], wg2_ref[...],
                          preferred_element_type=jnp.float32)


def _prep(x2d, W1, b1, W2, Wg1, Wg2, bg1):
    return pl.pallas_call(
        _prep_body,
        out_shape=(
            jax.ShapeDtypeStruct((_T * _N, 1), jnp.float32),
            jax.ShapeDtypeStruct((_T * _N, 1), jnp.float32),
            jax.ShapeDtypeStruct((1, 1), jnp.float32),
        ),
    )(x2d, W1, b1.reshape(1, -1), W2, Wg1, Wg2, bg1.reshape(1, -1))


_ROWS = 400


def _mv_body(ta_ref, v_ref, add_ref, o_ref, *, add_scale):
    acc = jnp.dot(ta_ref[...], v_ref[...], preferred_element_type=jnp.float32)
    o_ref[...] = acc / _TAU + add_ref[...] * add_scale


def _mv_pass(ta, v, add, add_scale, add_blocked):
    nblk = _N // _ROWS
    add_spec = (pl.BlockSpec((_ROWS, _T), lambda i: (i, 0)) if add_blocked
                else pl.BlockSpec(add.shape, lambda i: (0, 0)))
    return pl.pallas_call(
        functools.partial(_mv_body, add_scale=add_scale),
        grid=(nblk,),
        in_specs=[
            pl.BlockSpec((_ROWS, _N), lambda i: (i, 0)),
            pl.BlockSpec((_N, _T), lambda i: (0, 0)),
            add_spec,
        ],
        out_specs=pl.BlockSpec((_ROWS, _T), lambda i: (i, 0)),
        out_shape=jax.ShapeDtypeStruct((_N, _T), jnp.float32),
    )(ta, v, add)


def _rsqrt16(d):
    i = lax.bitcast_convert_type(d, jnp.int32)
    i = 0x5F3759DF - lax.shift_right_arithmetic(i, 1)
    y = lax.bitcast_convert_type(i, jnp.float32)
    for _ in range(4):
        y = y * (1.5 - 0.5 * d * y * y)
    return jnp.where(d > 0, y, 0.0)


def _gcn_body(r3_hbm, c3_hbm, w4_hbm, sg_hbm, cst_hbm, out_hbm,
              r1d, c1d, w1d, norm1d, vals1d,
              sg_full, dinv_full, xw2_full,
              sl_a, sl_b, sl_c, zer_sl, cst_loc,
              acc_sh, bcast_sh):
    core = lax.axis_index("c")
    sub = lax.axis_index("s")
    nbase = sub * _NSL
    own = pl.ds(nbase, _NSL)
    f32 = jnp.float32

    pltpu.sync_copy(r3_hbm.at[pl.ds(sub * _ECP, _ECP)], r1d)
    pltpu.sync_copy(c3_hbm.at[pl.ds(sub * _ECP, _ECP)], c1d)
    pltpu.sync_copy(cst_hbm, cst_loc)
    zeros16 = jnp.zeros((_L,), f32)

    def zer_body(i, cc):
        zer_sl[pl.ds(i * _L, _L)] = zeros16
        return cc
    lax.fori_loop(0, _NSL // _L, zer_body, 0)
    c1v = cst_loc[pl.ds(0, _L)]
    bg2v = cst_loc[pl.ds(_L, _L)]

    def do_t(t):
        pltpu.sync_copy(w4_hbm.at[pl.ds((t * _NSUB + sub) * _ECP, _ECP)], w1d)
        pltpu.sync_copy(sg_hbm.at[pl.ds(t * _NP, _NP)], sg_full)
        pltpu.sync_copy(zer_sl, acc_sh.at[own])
        plsc.subcore_barrier()
        pltpu.sync_copy(w1d, acc_sh.at[c1d], add=True)
        plsc.subcore_barrier()
        pltpu.sync_copy(acc_sh.at[own], sl_a)

        def dinv_body(i, cc):
            d = sl_a[pl.ds(i * _L, _L)] + 1.0
            sl_b[pl.ds(i * _L, _L)] = _rsqrt16(d)
            return cc
        lax.fori_loop(0, _NSL // _L, dinv_body, 0)
        pltpu.sync_copy(zer_sl, acc_sh.at[own])
        pltpu.sync_copy(sl_b, bcast_sh.at[own])
        plsc.subcore_barrier()
        pltpu.sync_copy(bcast_sh, dinv_full)

        def l1_body(j, cc):
            off = pl.ds(j * _L, _L)
            r16 = r1d[off]
            c16 = c1d[off]
            w16 = w1d[off]
            dr = plsc.load_gather(dinv_full, [r16])
            dc = plsc.load_gather(dinv_full, [c16])
            nrm = dr * w16 * dc
            norm1d[off] = nrm
            sv = plsc.load_gather(sg_full, [r16])
            vals1d[off] = nrm * sv
            return cc
        lax.fori_loop(0, _ECP // _L, l1_body, 0)
        plsc.subcore_barrier()
        pltpu.sync_copy(vals1d, acc_sh.at[c1d], add=True)
        plsc.subcore_barrier()
        pltpu.sync_copy(acc_sh.at[own], sl_a)

        def xw2_body(i, cc):
            sl16 = pl.ds(i * _L, _L)
            dv = sl_b[sl16]
            sg16 = sg_full[pl.ds(nbase + i * _L, _L)]
            sl_c[sl16] = sl_a[sl16] + dv * dv * sg16 + c1v
            return cc
        lax.fori_loop(0, _NSL // _L, xw2_body, 0)
        pltpu.sync_copy(zer_sl, acc_sh.at[own])
        pltpu.sync_copy(sl_c, bcast_sh.at[own])
        plsc.subcore_barrier()
        pltpu.sync_copy(bcast_sh, xw2_full)

        def l2_body(j, cc):
            off = pl.ds(j * _L, _L)
            r16 = r1d[off]
            nrm = norm1d[off]
            xv = plsc.load_gather(xw2_full, [r16])
            vals1d[off] = nrm * xv
            return cc
        lax.fori_loop(0, _ECP // _L, l2_body, 0)
        plsc.subcore_barrier()
        pltpu.sync_copy(vals1d, acc_sh.at[c1d], add=True)
        plsc.subcore_barrier()
        pltpu.sync_copy(acc_sh.at[own], sl_a)

        def g2_body(i, cc):
            sl16 = pl.ds(i * _L, _L)
            dv = sl_b[sl16]
            x16 = xw2_full[pl.ds(nbase + i * _L, _L)]
            sl_c[sl16] = sl_a[sl16] + dv * dv * x16 + bg2v
            return cc
        lax.fori_loop(0, _NSL // _L, g2_body, 0)
        pltpu.sync_copy(sl_c, out_hbm.at[pl.ds(t * _NP + nbase, _NSL)])
        plsc.subcore_barrier()

    do_t(core)

    @pl.when(core == 0)
    def _():
        do_t(2)


def _gcn(r3, c3, w4, sgp, cst):
    mesh = plsc.VectorSubcoreMesh(core_axis_name="c", subcore_axis_name="s")
    return pl.kernel(
        _gcn_body,
        out_type=jax.ShapeDtypeStruct((_T * _NP,), jnp.float32),
        mesh=mesh,
        compiler_params=pltpu.CompilerParams(needs_layout_passes=False),
        scratch_types=[
            pltpu.VMEM((_ECP,), jnp.int32),
            pltpu.VMEM((_ECP,), jnp.int32),
            pltpu.VMEM((_ECP,), jnp.float32),
            pltpu.VMEM((_ECP,), jnp.float32),
            pltpu.VMEM((_ECP,), jnp.float32),
            pltpu.VMEM((_NP,), jnp.float32),
            pltpu.VMEM((_NP,), jnp.float32),
            pltpu.VMEM((_NP,), jnp.float32),
            pltpu.VMEM((_NSL,), jnp.float32),
            pltpu.VMEM((_NSL,), jnp.float32),
            pltpu.VMEM((_NSL,), jnp.float32),
            pltpu.VMEM((_NSL,), jnp.float32),
            pltpu.VMEM((2 * _L,), jnp.float32),
            pltpu.VMEM_SHARED((_NP,), jnp.float32),
            pltpu.VMEM_SHARED((_NP,), jnp.float32),
        ],
    )(r3, c3, w4, sgp, cst)


def kernel(TSdata, time_adj, edge, edge_attr, W1, b1, W2, b2,
           Wg1, bg1, Wg2, bg2):
    x2d = TSdata.reshape(_T * _N, _D)
    st, sg, c1 = _prep(x2d, W1, b1, W2, Wg1, Wg2, bg1)

    S = st.reshape(_T, _N).T
    sgp = jnp.pad(sg.reshape(_T, _N), ((0, 0), (0, _NP - _N)))

    edge = edge.astype(jnp.int32)
    rp = jnp.pad(edge[0].reshape(_NSUB, _EC), ((0, 0), (0, _ECP - _EC)),
                 constant_values=_NP - 1)
    cp = jnp.pad(edge[1].reshape(_NSUB, _EC), ((0, 0), (0, _ECP - _EC)),
                 constant_values=_NP - 1)
    wp = jnp.pad(edge_attr.reshape(_T, _NSUB, _EC),
                 ((0, 0), (0, 0), (0, _ECP - _EC)))

    cst = jnp.concatenate([jnp.full((_L,), c1[0, 0], jnp.float32),
                           jnp.full((_L,), bg2[0], jnp.float32)])

    g2 = _gcn(rp.reshape(-1), cp.reshape(-1), wp.reshape(-1),
              sgp.reshape(-1), cst).reshape(_T, _NP)

    b2row = jnp.broadcast_to(b2.reshape(1, 1), (1, _T)).astype(jnp.float32)
    inner = _mv_pass(time_adj, S, b2row, 1.0, add_blocked=False)
    g2t = g2[:, :_N].T
    dense = _mv_pass(time_adj, inner, g2t, _BETA, add_blocked=True)

    return dense.T


# speedup vs baseline: 38.1266x; 1.0404x over previous
import functools

import jax
import jax.numpy as jnp
from jax import lax
from jax.experimental import pallas as pl
from jax.experimental.pallas import tpu as pltpu
from jax.experimental.pallas import tpu_sc as plsc

_N = 10000
_E = 160000
_T = 3
_D = 128
_TAU = 3.0
_BETA = 0.5

_L = 16
_NSUB = 16
_NSL = 640
_NP = _NSUB * _NSL
_EC = _E // _NSUB
_ECP = 10240
_ER = _ECP // 128


_PB = 6
_PR = _T * _N // _PB


def _prep_body(x_ref, w1_ref, b1_ref, w2_ref, wg1_ref, wg2_ref, bg1_ref,
               bg2_ref, st_ref, sg_ref, cst_ref):
    w12 = jnp.dot(w1_ref[...], w2_ref[...], preferred_element_type=jnp.float32)
    wg12 = jnp.dot(wg1_ref[...], wg2_ref[...],
                   preferred_element_type=jnp.float32)
    rhs = jnp.concatenate([w12, wg12], axis=1)
    out = jnp.dot(x_ref[...], rhs, preferred_element_type=jnp.float32)
    b1w2 = jnp.dot(b1_ref[...], w2_ref[...], preferred_element_type=jnp.float32)
    st_ref[...] = out[:, 0:1] + b1w2[0, 0]
    sg_ref[...] = out[:, 1:2]
    c1 = jnp.dot(bg1_ref[...], wg2_ref[...],
                 preferred_element_type=jnp.float32)[0, 0]
    cst_ref[...] = jnp.concatenate(
        [jnp.full((1, _L), c1, jnp.float32),
         jnp.full((1, _L), bg2_ref[0, 0], jnp.float32)], axis=1)


def _prep(x2d, W1, b1, W2, Wg1, Wg2, bg1, bg2):
    wspec = pl.BlockSpec(None, lambda i: (0, 0))
    return pl.pallas_call(
        _prep_body,
        grid=(_PB,),
        in_specs=[pl.BlockSpec((_PR, _D), lambda i: (i, 0))] + [wspec] * 7,
        out_specs=(
            pl.BlockSpec((_PR, 1), lambda i: (i, 0)),
            pl.BlockSpec((_PR, 1), lambda i: (i, 0)),
            pl.BlockSpec((1, 2 * _L), lambda i: (0, 0)),
        ),
        out_shape=(
            jax.ShapeDtypeStruct((_T * _N, 1), jnp.float32),
            jax.ShapeDtypeStruct((_T * _N, 1), jnp.float32),
            jax.ShapeDtypeStruct((1, 2 * _L), jnp.float32),
        ),
    )(x2d, W1, b1.reshape(1, -1), W2, Wg1, Wg2, bg1.reshape(1, -1),
      bg2.reshape(1, 1))


_ROWS = 400


def _mv_body(ta_ref, v_ref, add_ref, o_ref, *, add_scale):
    acc = jnp.dot(ta_ref[...], v_ref[...], preferred_element_type=jnp.float32)
    o_ref[...] = acc / _TAU + add_ref[...] * add_scale


def _mv_pass(ta, v, add, add_scale, add_blocked):
    nblk = _N // _ROWS
    add_spec = (pl.BlockSpec((_ROWS, _T), lambda i: (i, 0)) if add_blocked
                else pl.BlockSpec((1, 1), lambda i: (0, 0)))
    return pl.pallas_call(
        functools.partial(_mv_body, add_scale=add_scale),
        grid=(nblk,),
        in_specs=[
            pl.BlockSpec((_ROWS, _N), lambda i: (i, 0)),
            pl.BlockSpec((_N, _T), lambda i: (0, 0)),
            add_spec,
        ],
        out_specs=pl.BlockSpec((_ROWS, _T), lambda i: (i, 0)),
        out_shape=jax.ShapeDtypeStruct((_N, _T), jnp.float32),
    )(ta, v, add)


def _rsqrt16(d):
    i = lax.bitcast_convert_type(d, jnp.int32)
    i = 0x5F3759DF - lax.shift_right_arithmetic(i, 1)
    y = lax.bitcast_convert_type(i, jnp.float32)
    for _ in range(4):
        y = y * (1.5 - 0.5 * d * y * y)
    return jnp.where(d > 0, y, 0.0)


def _gcn_body(rc_hbm, w4_hbm, sg_hbm, cst_hbm, out_hbm,
              r1d, c1d, w1d, norm1d, vals1d,
              sg_full, dinv_full, xw2_full,
              sl_a, sl_b, sl_c, zer_sl, cst_loc,
              acc_sh, bcast_sh):
    core = lax.axis_index("c")
    sub = lax.axis_index("s")
    nbase = sub * _NSL
    own = pl.ds(nbase, _NSL)
    f32 = jnp.float32

    pltpu.sync_copy(rc_hbm.at[pl.ds(sub * _EC, _EC)], r1d)
    pltpu.sync_copy(rc_hbm.at[pl.ds(_E + sub * _EC, _EC)], c1d)
    pltpu.sync_copy(cst_hbm, cst_loc)
    zeros16 = jnp.zeros((_L,), f32)

    def zer_body(i, cc):
        zer_sl[pl.ds(i * _L, _L)] = zeros16
        return cc
    lax.fori_loop(0, _NSL // _L, zer_body, 0)
    c1v = cst_loc[pl.ds(0, _L)]
    bg2v = cst_loc[pl.ds(_L, _L)]

    def do_t(t):
        pltpu.sync_copy(w4_hbm.at[pl.ds(t * _E + sub * _EC, _EC)], w1d)
        pltpu.sync_copy(sg_hbm.at[pl.ds(t * _NP, _NP)], sg_full)
        pltpu.sync_copy(zer_sl, acc_sh.at[own])
        plsc.subcore_barrier()
        pltpu.sync_copy(w1d, acc_sh.at[c1d], add=True)
        plsc.subcore_barrier()
        pltpu.sync_copy(acc_sh.at[own], sl_a)

        def dinv_body(i, cc):
            d = sl_a[pl.ds(i * _L, _L)] + 1.0
            sl_b[pl.ds(i * _L, _L)] = _rsqrt16(d)
            return cc
        lax.fori_loop(0, _NSL // _L, dinv_body, 0)
        pltpu.sync_copy(zer_sl, acc_sh.at[own])
        pltpu.sync_copy(sl_b, bcast_sh.at[own])
        plsc.subcore_barrier()
        pltpu.sync_copy(bcast_sh, dinv_full)

        def l1_body(j, cc):
            off = pl.ds(j * _L, _L)
            r16 = r1d[off]
            c16 = c1d[off]
            w16 = w1d[off]
            dr = plsc.load_gather(dinv_full, [r16])
            dc = plsc.load_gather(dinv_full, [c16])
            nrm = dr * w16 * dc
            norm1d[off] = nrm
            sv = plsc.load_gather(sg_full, [r16])
            vals1d[off] = nrm * sv
            return cc
        lax.fori_loop(0, _EC // _L, l1_body, 0)
        plsc.subcore_barrier()
        pltpu.sync_copy(vals1d, acc_sh.at[c1d], add=True)
        plsc.subcore_barrier()
        pltpu.sync_copy(acc_sh.at[own], sl_a)

        def xw2_body(i, cc):
            sl16 = pl.ds(i * _L, _L)
            dv = sl_b[sl16]
            sg16 = sg_full[pl.ds(nbase + i * _L, _L)]
            sl_c[sl16] = sl_a[sl16] + dv * dv * sg16 + c1v
            return cc
        lax.fori_loop(0, _NSL // _L, xw2_body, 0)
        pltpu.sync_copy(zer_sl, acc_sh.at[own])
        pltpu.sync_copy(sl_c, bcast_sh.at[own])
        plsc.subcore_barrier()
        pltpu.sync_copy(bcast_sh, xw2_full)

        def l2_body(j, cc):
            off = pl.ds(j * _L, _L)
            r16 = r1d[off]
            nrm = norm1d[off]
            xv = plsc.load_gather(xw2_full, [r16])
            vals1d[off] = nrm * xv
            return cc
        lax.fori_loop(0, _EC // _L, l2_body, 0)
        plsc.subcore_barrier()
        pltpu.sync_copy(vals1d, acc_sh.at[c1d], add=True)
        plsc.subcore_barrier()
        pltpu.sync_copy(acc_sh.at[own], sl_a)

        def g2_body(i, cc):
            sl16 = pl.ds(i * _L, _L)
            dv = sl_b[sl16]
            x16 = xw2_full[pl.ds(nbase + i * _L, _L)]
            sl_c[sl16] = sl_a[sl16] + dv * dv * x16 + bg2v
            return cc
        lax.fori_loop(0, _NSL // _L, g2_body, 0)
        pltpu.sync_copy(sl_c, out_hbm.at[pl.ds(t * _NP + nbase, _NSL)])
        plsc.subcore_barrier()

    do_t(core)

    @pl.when(core == 0)
    def _():
        do_t(2)


def _gcn(rc, w4, sgp, cst):
    mesh = plsc.VectorSubcoreMesh(core_axis_name="c", subcore_axis_name="s")
    return pl.kernel(
        _gcn_body,
        out_type=jax.ShapeDtypeStruct((_T * _NP,), jnp.float32),
        mesh=mesh,
        compiler_params=pltpu.CompilerParams(needs_layout_passes=False),
        scratch_types=[
            pltpu.VMEM((_EC,), jnp.int32),
            pltpu.VMEM((_EC,), jnp.int32),
            pltpu.VMEM((_EC,), jnp.float32),
            pltpu.VMEM((_EC,), jnp.float32),
            pltpu.VMEM((_EC,), jnp.float32),
            pltpu.VMEM((_NP,), jnp.float32),
            pltpu.VMEM((_NP,), jnp.float32),
            pltpu.VMEM((_NP,), jnp.float32),
            pltpu.VMEM((_NSL,), jnp.float32),
            pltpu.VMEM((_NSL,), jnp.float32),
            pltpu.VMEM((_NSL,), jnp.float32),
            pltpu.VMEM((_NSL,), jnp.float32),
            pltpu.VMEM((2 * _L,), jnp.float32),
            pltpu.VMEM_SHARED((_NP,), jnp.float32),
            pltpu.VMEM_SHARED((_NP,), jnp.float32),
        ],
    )(rc, w4, sgp, cst)


def kernel(TSdata, time_adj, edge, edge_attr, W1, b1, W2, b2,
           Wg1, bg1, Wg2, bg2):
    x2d = TSdata.reshape(_T * _N, _D)
    st, sg, cst = _prep(x2d, W1, b1, W2, Wg1, Wg2, bg1, bg2)

    S = st.reshape(_T, _N).T
    sgp = jnp.pad(sg.reshape(_T, _N), ((0, 0), (0, _NP - _N)))

    g2 = _gcn(edge.astype(jnp.int32).reshape(-1), edge_attr.reshape(-1),
              sgp.reshape(-1), cst.reshape(-1)).reshape(_T, _NP)

    inner = _mv_pass(time_adj, S, b2.reshape(1, 1), 1.0, add_blocked=False)
    g2t = g2[:, :_N].T
    dense = _mv_pass(time_adj, inner, g2t, _BETA, add_blocked=True)

    return dense.T


# speedup vs baseline: 39.2503x; 1.0295x over previous
import functools

import jax
import jax.numpy as jnp
from jax import lax
from jax.experimental import pallas as pl
from jax.experimental.pallas import tpu as pltpu
from jax.experimental.pallas import tpu_sc as plsc

_N = 10000
_E = 160000
_T = 3
_D = 128
_TAU = 3.0
_BETA = 0.5

_L = 16
_NSUB = 16
_NSL = 640
_NP = _NSUB * _NSL
_EC = _E // _NSUB
_ECP = 10240
_ER = _ECP // 128


_NB = 2048
_HI = jax.lax.Precision.HIGHEST


def _dot3(a, b):
    f = jnp.float32
    ah = a.astype(jnp.bfloat16)
    al = (a - ah.astype(f)).astype(jnp.bfloat16)
    bh = b.astype(jnp.bfloat16)
    bl = (b - bh.astype(f)).astype(jnp.bfloat16)
    return (jnp.dot(ah, bh, preferred_element_type=f)
            + (jnp.dot(ah, bl, preferred_element_type=f)
               + jnp.dot(al, bh, preferred_element_type=f)))


def _prep_body(x_ref, w1_ref, b1_ref, w2_ref, wg1_ref, wg2_ref, bg1_ref,
               bg2_ref, sh_ref, sl_ref, sgt_ref, cst_ref):
    w12 = jnp.dot(w1_ref[...], w2_ref[...], precision=_HI,
                  preferred_element_type=jnp.float32)
    wg12 = jnp.dot(wg1_ref[...], wg2_ref[...], precision=_HI,
                   preferred_element_type=jnp.float32)
    rhs = jnp.concatenate([w12, wg12], axis=1)
    b1w2 = jnp.dot(b1_ref[...], w2_ref[...], precision=_HI,
                   preferred_element_type=jnp.float32)[0, 0]
    rh, rl = _split(rhs)
    st_cols = []
    sg_cols = []
    for t in range(_T):
        xh = x_ref[t].astype(jnp.bfloat16)
        out = (jnp.dot(xh, rh, preferred_element_type=jnp.float32)
               + jnp.dot(xh, rl, preferred_element_type=jnp.float32))
        st_cols.append(out[:, 0:1] + b1w2)
        sg_cols.append(out[:, 1:2])
    s_blk = jnp.concatenate(st_cols, axis=1)
    sh = s_blk.astype(jnp.bfloat16)
    sh_ref[...] = sh
    sl_ref[...] = (s_blk - sh.astype(jnp.float32)).astype(jnp.bfloat16)
    sgt_ref[...] = jnp.concatenate(sg_cols, axis=1).T
    c1 = jnp.dot(bg1_ref[...], wg2_ref[...], precision=_HI,
                 preferred_element_type=jnp.float32)[0, 0]
    cst_ref[...] = jnp.concatenate(
        [jnp.full((1, _L), c1, jnp.float32),
         jnp.full((1, _L), bg2_ref[0, 0], jnp.float32)], axis=1)


def _prep(x3d, W1, b1, W2, Wg1, Wg2, bg1, bg2):
    wspec = pl.BlockSpec(None, lambda i: (0, 0))
    return pl.pallas_call(
        _prep_body,
        grid=(pl.cdiv(_N, _NB),),
        in_specs=[pl.BlockSpec((_T, _NB, _D), lambda i: (0, i, 0))] +
                 [wspec] * 7,
        out_specs=(
            pl.BlockSpec((_NB, _T), lambda i: (i, 0)),
            pl.BlockSpec((_NB, _T), lambda i: (i, 0)),
            pl.BlockSpec((_T, _NB), lambda i: (0, i)),
            pl.BlockSpec((1, 2 * _L), lambda i: (0, 0)),
        ),
        out_shape=(
            jax.ShapeDtypeStruct((_N, _T), jnp.bfloat16),
            jax.ShapeDtypeStruct((_N, _T), jnp.bfloat16),
            jax.ShapeDtypeStruct((_T, _N), jnp.float32),
            jax.ShapeDtypeStruct((1, 2 * _L), jnp.float32),
        ),
    )(x3d, W1, b1.reshape(1, -1), W2, Wg1, Wg2, bg1.reshape(1, -1),
      bg2.reshape(1, 1))


_ROWS = 512


def _split(x):
    hi = x.astype(jnp.bfloat16)
    lo = (x - hi.astype(jnp.float32)).astype(jnp.bfloat16)
    return hi, lo


def _dot3p(ta, vh, vl):
    f = jnp.float32
    th = ta.astype(jnp.bfloat16)
    return (jnp.dot(th, vh, preferred_element_type=f)
            + jnp.dot(th, vl, preferred_element_type=f))


def _mv_a_body(ta_ref, vh_ref, vl_ref, b2_ref, ih_ref, il_ref):
    acc = _dot3p(ta_ref[...], vh_ref[...], vl_ref[...])
    inner = acc / _TAU + b2_ref[0, 0]
    ih, il = _split(inner)
    ih_ref[...] = ih
    il_ref[...] = il


def _mv_b_body(ta_ref, vh_ref, vl_ref, g2_ref, o_ref):
    acc = _dot3p(ta_ref[...], vh_ref[...], vl_ref[...])
    o_ref[...] = acc.T / _TAU + g2_ref[...] * _BETA


def _mv_pass_a(ta, vh, vl, b2r):
    return pl.pallas_call(
        _mv_a_body,
        grid=(pl.cdiv(_N, _ROWS),),
        in_specs=[
            pl.BlockSpec((_ROWS, _N), lambda i: (i, 0)),
            pl.BlockSpec((_N, _T), lambda i: (0, 0)),
            pl.BlockSpec((_N, _T), lambda i: (0, 0)),
            pl.BlockSpec((1, 1), lambda i: (0, 0)),
        ],
        out_specs=(pl.BlockSpec((_ROWS, _T), lambda i: (i, 0)),
                   pl.BlockSpec((_ROWS, _T), lambda i: (i, 0))),
        out_shape=(jax.ShapeDtypeStruct((_N, _T), jnp.bfloat16),
                   jax.ShapeDtypeStruct((_N, _T), jnp.bfloat16)),
    )(ta, vh, vl, b2r)


def _mv_pass_b(ta, vh, vl, g2):
    return pl.pallas_call(
        _mv_b_body,
        grid=(pl.cdiv(_N, _ROWS),),
        in_specs=[
            pl.BlockSpec((_ROWS, _N), lambda i: (i, 0)),
            pl.BlockSpec((_N, _T), lambda i: (0, 0)),
            pl.BlockSpec((_N, _T), lambda i: (0, 0)),
            pl.BlockSpec((_T, _ROWS), lambda i: (0, i)),
        ],
        out_specs=pl.BlockSpec((_T, _ROWS), lambda i: (0, i)),
        out_shape=jax.ShapeDtypeStruct((_T, _N), jnp.float32),
    )(ta, vh, vl, g2)


def _rsqrt16(d):
    i = lax.bitcast_convert_type(d, jnp.int32)
    i = 0x5F3759DF - lax.shift_right_arithmetic(i, 1)
    y = lax.bitcast_convert_type(i, jnp.float32)
    for _ in range(4):
        y = y * (1.5 - 0.5 * d * y * y)
    return jnp.where(d > 0, y, 0.0)


def _gcn_body(rc_hbm, w4_hbm, sg_hbm, cst_hbm, out_hbm,
              r1d, c1d, w1d, norm1d, vals1d,
              sg_full, dinv_full, xw2_full,
              sl_a, sl_b, sl_c, zer_sl, cst_loc,
              acc_sh, bcast_sh):
    core = lax.axis_index("c")
    sub = lax.axis_index("s")
    nbase = sub * _NSL
    own = pl.ds(nbase, _NSL)
    f32 = jnp.float32

    pltpu.sync_copy(rc_hbm.at[pl.ds(sub * _EC, _EC)], r1d)
    pltpu.sync_copy(rc_hbm.at[pl.ds(_E + sub * _EC, _EC)], c1d)
    pltpu.sync_copy(cst_hbm, cst_loc)
    zeros16 = jnp.zeros((_L,), f32)

    def zer_body(i, cc):
        zer_sl[pl.ds(i * _L, _L)] = zeros16
        return cc
    lax.fori_loop(0, _NSL // _L, zer_body, 0)
    c1v = cst_loc[pl.ds(0, _L)]
    bg2v = cst_loc[pl.ds(_L, _L)]

    def do_t(t):
        pltpu.sync_copy(w4_hbm.at[pl.ds(t * _E + sub * _EC, _EC)], w1d)
        pltpu.sync_copy(sg_hbm.at[pl.ds(t * _N, _N)], sg_full.at[pl.ds(0, _N)])
        pltpu.sync_copy(zer_sl, acc_sh.at[own])
        plsc.subcore_barrier()
        pltpu.sync_copy(w1d, acc_sh.at[c1d], add=True)
        plsc.subcore_barrier()
        pltpu.sync_copy(acc_sh.at[own], sl_a)

        def dinv_body(i, cc):
            d = sl_a[pl.ds(i * _L, _L)] + 1.0
            sl_b[pl.ds(i * _L, _L)] = _rsqrt16(d)
            return cc
        lax.fori_loop(0, _NSL // _L, dinv_body, 0)
        pltpu.sync_copy(zer_sl, acc_sh.at[own])
        pltpu.sync_copy(sl_b, bcast_sh.at[own])
        plsc.subcore_barrier()
        pltpu.sync_copy(bcast_sh, dinv_full)

        def l1_body(j, cc):
            off = pl.ds(j * _L, _L)
            r16 = r1d[off]
            c16 = c1d[off]
            w16 = w1d[off]
            dr = plsc.load_gather(dinv_full, [r16])
            dc = plsc.load_gather(dinv_full, [c16])
            nrm = dr * w16 * dc
            norm1d[off] = nrm
            sv = plsc.load_gather(sg_full, [r16])
            vals1d[off] = nrm * sv
            return cc
        lax.fori_loop(0, _EC // _L, l1_body, 0)
        plsc.subcore_barrier()
        pltpu.sync_copy(vals1d, acc_sh.at[c1d], add=True)
        plsc.subcore_barrier()
        pltpu.sync_copy(acc_sh.at[own], sl_a)

        def xw2_body(i, cc):
            sl16 = pl.ds(i * _L, _L)
            dv = sl_b[sl16]
            sg16 = sg_full[pl.ds(nbase + i * _L, _L)]
            sl_c[sl16] = sl_a[sl16] + dv * dv * sg16 + c1v
            return cc
        lax.fori_loop(0, _NSL // _L, xw2_body, 0)
        pltpu.sync_copy(zer_sl, acc_sh.at[own])
        pltpu.sync_copy(sl_c, bcast_sh.at[own])
        plsc.subcore_barrier()
        pltpu.sync_copy(bcast_sh, xw2_full)

        def l2_body(j, cc):
            off = pl.ds(j * _L, _L)
            r16 = r1d[off]
            nrm = norm1d[off]
            xv = plsc.load_gather(xw2_full, [r16])
            vals1d[off] = nrm * xv
            return cc
        lax.fori_loop(0, _EC // _L, l2_body, 0)
        plsc.subcore_barrier()
        pltpu.sync_copy(vals1d, acc_sh.at[c1d], add=True)
        plsc.subcore_barrier()
        pltpu.sync_copy(acc_sh.at[own], sl_a)

        def g2_body(i, cc):
            sl16 = pl.ds(i * _L, _L)
            dv = sl_b[sl16]
            x16 = xw2_full[pl.ds(nbase + i * _L, _L)]
            sl_c[sl16] = sl_a[sl16] + dv * dv * x16 + bg2v
            return cc
        lax.fori_loop(0, _NSL // _L, g2_body, 0)
        pltpu.sync_copy(sl_c, out_hbm.at[pl.ds(t * _NP + nbase, _NSL)])
        plsc.subcore_barrier()

    do_t(core)

    @pl.when(core == 0)
    def _():
        do_t(2)


def _gcn(rc, w4, sgp, cst):
    mesh = plsc.VectorSubcoreMesh(core_axis_name="c", subcore_axis_name="s")
    return pl.kernel(
        _gcn_body,
        out_type=jax.ShapeDtypeStruct((_T * _NP,), jnp.float32),
        mesh=mesh,
        compiler_params=pltpu.CompilerParams(needs_layout_passes=False),
        scratch_types=[
            pltpu.VMEM((_EC,), jnp.int32),
            pltpu.VMEM((_EC,), jnp.int32),
            pltpu.VMEM((_EC,), jnp.float32),
            pltpu.VMEM((_EC,), jnp.float32),
            pltpu.VMEM((_EC,), jnp.float32),
            pltpu.VMEM((_NP,), jnp.float32),
            pltpu.VMEM((_NP,), jnp.float32),
            pltpu.VMEM((_NP,), jnp.float32),
            pltpu.VMEM((_NSL,), jnp.float32),
            pltpu.VMEM((_NSL,), jnp.float32),
            pltpu.VMEM((_NSL,), jnp.float32),
            pltpu.VMEM((_NSL,), jnp.float32),
            pltpu.VMEM((2 * _L,), jnp.float32),
            pltpu.VMEM_SHARED((_NP,), jnp.float32),
            pltpu.VMEM_SHARED((_NP,), jnp.float32),
        ],
    )(rc, w4, sgp, cst)


def kernel(TSdata, time_adj, edge, edge_attr, W1, b1, W2, b2,
           Wg1, bg1, Wg2, bg2):
    Sh, Sl, sgT, cst = _prep(TSdata, W1, b1, W2, Wg1, Wg2, bg1, bg2)

    g2 = _gcn(edge.astype(jnp.int32).reshape(-1), edge_attr.reshape(-1),
              sgT.reshape(-1), cst.reshape(-1)).reshape(_T, _NP)

    ih, il = _mv_pass_a(time_adj, Sh, Sl, b2.reshape(1, 1))
    return _mv_pass_b(time_adj, ih, il, g2)
